# Initial kernel scaffold; baseline (speedup 1.0000x reference)
import jax
import jax.numpy as jnp
from jax.experimental import pallas as pl


def kernel(x, edge_index, edge_attr, params):
    raise NotImplementedError("write your pallas kernel here")


# speedup vs baseline: 1.5912x; 1.5912x over previous
import functools

import jax
import jax.numpy as jnp
from jax import lax
from jax.experimental import pallas as pl
from jax.experimental.pallas import tpu as pltpu
from jax.experimental.pallas import tpu_sc as plsc

N = 10000
E = 320000
E2 = E // 2
D = 128
DE = 16
H = 64
H2 = 2 * H
EPS = 1e-5

NC = 2
NS = 16
NW = NC * NS
EW = E // NW
CW = 80
CP = CW // 2
CH = EW // CW
NP = 10240
SRS = NP // NS
CE = 8000
GE = E2 // CE

_f32 = jnp.float32


def _swish(x):
    return x * (1.0 / (1.0 + jnp.exp(-x)))


def _mm_t(x, w):
    return lax.dot_general(x, w, (((1,), (1,)), ((), ())),
                           preferred_element_type=_f32)


def _bdiag(w):
    o, i = w.shape
    z = jnp.zeros((o, i), w.dtype)
    return jnp.concatenate(
        [jnp.concatenate([w, z], axis=1), jnp.concatenate([z, w], axis=1)],
        axis=0)


def _tile2(v):
    return jnp.concatenate([v, v]).reshape(1, -1)


def _bn_affine_paired(stats, g, be):
    ssum = stats[0:1, :H] + stats[0:1, H:]
    ssq = stats[1:2, :H] + stats[1:2, H:]
    m = ssum / E
    v = ssq / E - m * m
    scale = g[:, :H] / jnp.sqrt(v + EPS)
    shift = be[:, :H] - m * scale
    scale2 = jnp.concatenate([scale, scale], axis=1)
    shift2 = jnp.concatenate([shift, shift], axis=1)
    return scale2, shift2


def _head_body(x_ref, w_ref, b_ref, g_ref, be_ref, pwd_ref, pbd_ref, pws_ref,
               node_ref, t_ref):
    h = _swish(_mm_t(x_ref[...], w_ref[...]) + b_ref[...])
    m = jnp.mean(h, axis=0, keepdims=True)
    v = jnp.mean((h - m) * (h - m), axis=0, keepdims=True)
    node = g_ref[...] * (h - m) / jnp.sqrt(v + EPS) + be_ref[...]
    node_ref[...] = node
    ta = _mm_t(node, pwd_ref[...]) + pbd_ref[...]
    tb = _mm_t(node, pws_ref[...])
    t_ref[...] = jnp.concatenate([ta, tb], axis=1)


def _head(x, w, b, g, be, pwd, pbd, pws):
    out = [jax.ShapeDtypeStruct((N, H), _f32),
           jax.ShapeDtypeStruct((N, H2), _f32)]
    return pl.pallas_call(_head_body, out_shape=out)(
        x, w, b, g, be, pwd, pbd, pws)


def _edge_u_body(ea_ref, w_ref, b_ref, u_ref, st_ref, acc_ref):
    i = pl.program_id(0)
    s = _swish(_mm_t(ea_ref[...], w_ref[...]) + b_ref[...])
    u_ref[...] = s

    @pl.when(i == 0)
    def _():
        acc_ref[...] = jnp.zeros_like(acc_ref)

    acc_ref[0:1, :] += jnp.sum(s, axis=0, keepdims=True)
    acc_ref[1:2, :] += jnp.sum(s * s, axis=0, keepdims=True)

    @pl.when(i == pl.num_programs(0) - 1)
    def _():
        st_ref[...] = acc_ref[...]


def _edge_u(ea2, w2, b2):
    return pl.pallas_call(
        _edge_u_body,
        grid=(GE,),
        in_specs=[
            pl.BlockSpec((CE, 2 * DE), lambda i: (i, 0)),
            pl.BlockSpec((H2, 2 * DE), lambda i: (0, 0)),
            pl.BlockSpec((1, H2), lambda i: (0, 0)),
        ],
        out_specs=[
            pl.BlockSpec((CE, H2), lambda i: (i, 0)),
            pl.BlockSpec((2, H2), lambda i: (0, 0)),
        ],
        out_shape=[
            jax.ShapeDtypeStruct((E2, H2), _f32),
            jax.ShapeDtypeStruct((2, H2), _f32),
        ],
        scratch_shapes=[pltpu.VMEM((2, H2), _f32)],
    )(ea2, w2, b2)


def _msg_body(s_ref, st_ref, w_ref, b_ref, g_ref, be_ref, out_ref):
    stats = jnp.sum(st_ref[...], axis=0)
    m = stats[0:1, :] / E
    v = stats[1:2, :] / E - m * m
    scale = g_ref[...] / jnp.sqrt(v + EPS)
    shift = be_ref[...] - m * scale
    scale2 = jnp.concatenate([scale, scale], axis=1)
    shift2 = jnp.concatenate([shift, shift], axis=1)
    sn = s_ref[...] * scale2 + shift2
    out_ref[...] = _swish(_mm_t(sn, w_ref[...]) + b_ref[...])


def _msg(s, st, w2, b2, g, be):
    return pl.pallas_call(
        _msg_body,
        grid=(GE,),
        in_specs=[
            pl.BlockSpec((CE, H2), lambda i: (i, 0)),
            pl.BlockSpec((NW, 2, H), lambda i: (0, 0, 0)),
            pl.BlockSpec((H2, H2), lambda i: (0, 0)),
            pl.BlockSpec((1, H2), lambda i: (0, 0)),
            pl.BlockSpec((1, H), lambda i: (0, 0)),
            pl.BlockSpec((1, H), lambda i: (0, 0)),
        ],
        out_specs=pl.BlockSpec((CE, H2), lambda i: (i, 0)),
        out_shape=jax.ShapeDtypeStruct((E2, H2), _f32),
    )(s, st, w2, b2, g, be)


def _update_body(node_ref, aggp_ref, cnt_ref, uw1a_ref, uw1b_ref, ub1_ref,
                 uw2_ref, ub2_ref, pwd_ref, pbd_ref, pws_ref,
                 nn_ref, t_ref):
    node = node_ref[...]
    agg = aggp_ref[0, :N, :H] + aggp_ref[1, :N, :H]
    deg = cnt_ref[0, :N, 0:1] + cnt_ref[1, :N, 0:1]
    mean = agg / jnp.maximum(deg, 1.0)
    upd = _swish(_mm_t(node, uw1a_ref[...]) + _mm_t(mean, uw1b_ref[...])
                 + ub1_ref[...])
    upd = _swish(_mm_t(upd, uw2_ref[...]) + ub2_ref[...])
    nn = node + upd
    nn_ref[...] = nn
    ta = _mm_t(nn, pwd_ref[...]) + pbd_ref[...]
    tb = _mm_t(nn, pws_ref[...])
    t_ref[...] = jnp.concatenate([ta, tb], axis=1)


def _update(node, aggp, cnt, uw1a, uw1b, ub1, uw2, ub2, pwd, pbd, pws):
    out = [jax.ShapeDtypeStruct((N, H), _f32),
           jax.ShapeDtypeStruct((N, H2), _f32)]
    return pl.pallas_call(_update_body, out_shape=out)(
        node, aggp, cnt, uw1a, uw1b, ub1, uw2, ub2, pwd, pbd, pws)


def _t1_body(d_ref, u_ref, ste_ref, eg_ref, ebe_ref, w_ref, b_ref,
             t1_ref, st_ref, acc_ref):
    i = pl.program_id(0)
    scale2, shift2 = _bn_affine_paired(ste_ref[...], eg_ref[...], ebe_ref[...])
    edge = u_ref[...] * scale2 + shift2
    s = _swish(d_ref[...] + _mm_t(edge, w_ref[...]) + b_ref[...])
    t1_ref[...] = s

    @pl.when(i == 0)
    def _():
        acc_ref[...] = jnp.zeros_like(acc_ref)

    acc_ref[0:1, :] += jnp.sum(s, axis=0, keepdims=True)
    acc_ref[1:2, :] += jnp.sum(s * s, axis=0, keepdims=True)

    @pl.when(i == pl.num_programs(0) - 1)
    def _():
        st_ref[...] = acc_ref[...]


def _t1(d, u, ste, eg2, ebe2, w2, b2):
    return pl.pallas_call(
        _t1_body,
        grid=(GE,),
        in_specs=[
            pl.BlockSpec((CE, H2), lambda i: (i, 0)),
            pl.BlockSpec((CE, H2), lambda i: (i, 0)),
            pl.BlockSpec((2, H2), lambda i: (0, 0)),
            pl.BlockSpec((1, H2), lambda i: (0, 0)),
            pl.BlockSpec((1, H2), lambda i: (0, 0)),
            pl.BlockSpec((H2, H2), lambda i: (0, 0)),
            pl.BlockSpec((1, H2), lambda i: (0, 0)),
        ],
        out_specs=[
            pl.BlockSpec((CE, H2), lambda i: (i, 0)),
            pl.BlockSpec((2, H2), lambda i: (0, 0)),
        ],
        out_shape=[
            jax.ShapeDtypeStruct((E2, H2), _f32),
            jax.ShapeDtypeStruct((2, H2), _f32),
        ],
        scratch_shapes=[pltpu.VMEM((2, H2), _f32)],
    )(d, u, ste, eg2, ebe2, w2, b2)


def _t2_body(t1_ref, st1_ref, g1_ref, be1_ref, w_ref, b_ref,
             s2_ref, st_ref, acc_ref):
    i = pl.program_id(0)
    scale2, shift2 = _bn_affine_paired(st1_ref[...], g1_ref[...], be1_ref[...])
    tn = t1_ref[...] * scale2 + shift2
    s = _swish(_mm_t(tn, w_ref[...]) + b_ref[...])
    s2_ref[...] = s

    @pl.when(i == 0)
    def _():
        acc_ref[...] = jnp.zeros_like(acc_ref)

    acc_ref[0:1, :] += jnp.sum(s, axis=0, keepdims=True)
    acc_ref[1:2, :] += jnp.sum(s * s, axis=0, keepdims=True)

    @pl.when(i == pl.num_programs(0) - 1)
    def _():
        st_ref[...] = acc_ref[...]


def _t2(t1, st1, g12, be12, w2, b2):
    return pl.pallas_call(
        _t2_body,
        grid=(GE,),
        in_specs=[
            pl.BlockSpec((CE, H2), lambda i: (i, 0)),
            pl.BlockSpec((2, H2), lambda i: (0, 0)),
            pl.BlockSpec((1, H2), lambda i: (0, 0)),
            pl.BlockSpec((1, H2), lambda i: (0, 0)),
            pl.BlockSpec((H2, H2), lambda i: (0, 0)),
            pl.BlockSpec((1, H2), lambda i: (0, 0)),
        ],
        out_specs=[
            pl.BlockSpec((CE, H2), lambda i: (i, 0)),
            pl.BlockSpec((2, H2), lambda i: (0, 0)),
        ],
        out_shape=[
            jax.ShapeDtypeStruct((E2, H2), _f32),
            jax.ShapeDtypeStruct((2, H2), _f32),
        ],
        scratch_shapes=[pltpu.VMEM((2, H2), _f32)],
    )(t1, st1, g12, be12, w2, b2)


def _finout_body(s2_ref, st2_ref, g2_ref, be2_ref, y_ref):
    scale2, shift2 = _bn_affine_paired(st2_ref[...], g2_ref[...], be2_ref[...])
    y_ref[...] = s2_ref[...] * scale2 + shift2


def _finout(s2, st2, g22, be22):
    return pl.pallas_call(
        _finout_body,
        grid=(GE,),
        in_specs=[
            pl.BlockSpec((CE, H2), lambda i: (i, 0)),
            pl.BlockSpec((2, H2), lambda i: (0, 0)),
            pl.BlockSpec((1, H2), lambda i: (0, 0)),
            pl.BlockSpec((1, H2), lambda i: (0, 0)),
        ],
        out_specs=pl.BlockSpec((CE, H2), lambda i: (i, 0)),
        out_shape=jax.ShapeDtypeStruct((E2, H2), _f32),
    )(s2, st2, g22, be22)


@functools.cache
def _mesh():
    return plsc.VectorSubcoreMesh(core_axis_name="c", subcore_axis_name="s",
                                  num_cores=NC, num_subcores=NS)


def _worker_id():
    return lax.axis_index("s") * NC + lax.axis_index("c")


def _gather_swish_body(t_hbm, dst_hbm, src_hbm, s_hbm, st_hbm,
                       idxd, idxs, ra, rb, so, stv, sem):
    wid = _worker_id()
    pltpu.sync_copy(dst_hbm.at[wid], idxd)
    pltpu.sync_copy(src_hbm.at[wid], idxs)
    zz = jnp.zeros((16,), _f32)
    for q in range(8):
        stv[q] = zz

    def chunk(j, carry):
        cpa = pltpu.async_copy(t_hbm.at[idxd.at[j]], ra, sem)
        cpb = pltpu.async_copy(t_hbm.at[idxs.at[j]], rb, sem)
        cpa.wait()
        cpb.wait()

        def pair(p, c2):
            for half in range(2):
                e = 2 * p + half
                for q in range(4):
                    h = (ra[e, pl.ds(q * 16, 16)]
                         + rb[e, pl.ds(H + q * 16, 16)])
                    v = h * (1.0 / (1.0 + jnp.exp(-h)))
                    so[p, pl.ds(half * H + q * 16, 16)] = v
                    plsc.addupdate(stv.at[q], v)
                    plsc.addupdate(stv.at[4 + q], v * v)
            return c2

        lax.fori_loop(0, CP, pair, 0)
        pltpu.sync_copy(so, s_hbm.at[pl.ds(wid * (EW // 2) + j * CP, CP), :])
        return carry

    lax.fori_loop(0, CH, chunk, 0)
    pltpu.sync_copy(stv, st_hbm.at[wid])


@functools.cache
def _build_gather_swish():
    return pl.kernel(
        _gather_swish_body,
        out_type=[
            jax.ShapeDtypeStruct((E2, H2), _f32),
            jax.ShapeDtypeStruct((NW, 8, 16), _f32),
        ],
        mesh=_mesh(),
        scratch_types=[
            pltpu.VMEM((CH, CW), jnp.int32),
            pltpu.VMEM((CH, CW), jnp.int32),
            pltpu.VMEM((CW, H2), _f32),
            pltpu.VMEM((CW, H2), _f32),
            pltpu.VMEM((CP, H2), _f32),
            pltpu.VMEM((8, 16), _f32),
            pltpu.SemaphoreType.DMA,
        ],
    )


def _gather_swish(t, dst3, src3):
    return _build_gather_swish()(t, dst3, src3)


def _gather_diff_body(t_hbm, dst_hbm, src_hbm, d_hbm,
                      idxd, idxs, ra, rb, so, sem):
    wid = _worker_id()
    pltpu.sync_copy(dst_hbm.at[wid], idxd)
    pltpu.sync_copy(src_hbm.at[wid], idxs)

    def chunk(j, carry):
        cpa = pltpu.async_copy(t_hbm.at[idxd.at[j]], ra, sem)
        cpb = pltpu.async_copy(t_hbm.at[idxs.at[j]], rb, sem)
        cpa.wait()
        cpb.wait()

        def pair(p, c2):
            for half in range(2):
                e = 2 * p + half
                for q in range(4):
                    v = (ra[e, pl.ds(q * 16, 16)]
                         - rb[e, pl.ds(H + q * 16, 16)])
                    so[p, pl.ds(half * H + q * 16, 16)] = v
            return c2

        lax.fori_loop(0, CP, pair, 0)
        pltpu.sync_copy(so, d_hbm.at[pl.ds(wid * (EW // 2) + j * CP, CP), :])
        return carry

    lax.fori_loop(0, CH, chunk, 0)


@functools.cache
def _build_gather_diff():
    return pl.kernel(
        _gather_diff_body,
        out_type=jax.ShapeDtypeStruct((E2, H2), _f32),
        mesh=_mesh(),
        scratch_types=[
            pltpu.VMEM((CH, CW), jnp.int32),
            pltpu.VMEM((CH, CW), jnp.int32),
            pltpu.VMEM((CW, H2), _f32),
            pltpu.VMEM((CW, H2), _f32),
            pltpu.VMEM((CP, H2), _f32),
            pltpu.SemaphoreType.DMA,
        ],
    )


def _gather_diff(t, dst3, src3):
    return _build_gather_diff()(t, dst3, src3)


def _scatter_add_body(msg, dst3, aggp, idxd, mb, mb2, agg_sh):
    cid = lax.axis_index("c")
    sid = lax.axis_index("s")
    wid = sid * NC + cid
    pltpu.sync_copy(dst3.at[wid], idxd)

    zz = jnp.zeros((16,), _f32)

    def zrow(r, c):
        for q in range(8):
            mb2[r, pl.ds(q * 16, 16)] = zz
        return c

    lax.fori_loop(0, CW, zrow, 0)
    for t in range(SRS // CW):
        pltpu.sync_copy(mb2, agg_sh.at[pl.ds(sid * SRS + t * CW, CW), :])
    plsc.subcore_barrier()

    def chunk(j, carry):
        pltpu.sync_copy(msg.at[pl.ds(wid * (EW // 2) + j * CP, CP), :], mb)

        def pair(p, c2):
            for q in range(4):
                mb2[2 * p, pl.ds(q * 16, 16)] = mb[p, pl.ds(q * 16, 16)]
                mb2[2 * p + 1, pl.ds(q * 16, 16)] = mb[p, pl.ds(H + q * 16, 16)]
            return c2

        lax.fori_loop(0, CP, pair, 0)
        pltpu.sync_copy(mb2, agg_sh.at[idxd.at[j]], add=True)
        return carry

    lax.fori_loop(0, CH, chunk, 0)
    plsc.subcore_barrier()
    for t in range(SRS // CW):
        rows = pl.ds(sid * SRS + t * CW, CW)
        pltpu.sync_copy(agg_sh.at[rows, :], aggp.at[cid, rows, :])


@functools.cache
def _build_scatter_add():
    return pl.kernel(
        _scatter_add_body,
        out_type=jax.ShapeDtypeStruct((NC, NP, H2), _f32),
        mesh=_mesh(),
        scratch_types=[
            pltpu.VMEM((CH, CW), jnp.int32),
            pltpu.VMEM((CP, H2), _f32),
            pltpu.VMEM((CW, H2), _f32),
            pltpu.VMEM_SHARED((NP, H2), _f32),
        ],
    )


def _scatter_add(msg, dst3):
    return _build_scatter_add()(msg, dst3)


def _count_body(dst3, cntp, idxd, ones_b, zb, cnt_sh):
    cid = lax.axis_index("c")
    sid = lax.axis_index("s")
    wid = sid * NC + cid
    pltpu.sync_copy(dst3.at[wid], idxd)

    zz = jnp.zeros((16,), _f32)
    oo = jnp.ones((16,), _f32)

    def frow(r, c):
        for q in range(8):
            zb[r, pl.ds(q * 16, 16)] = zz
            ones_b[r, pl.ds(q * 16, 16)] = oo
        return c

    lax.fori_loop(0, CW, frow, 0)
    for t in range(SRS // CW):
        pltpu.sync_copy(zb, cnt_sh.at[pl.ds(sid * SRS + t * CW, CW), :])
    plsc.subcore_barrier()

    def chunk(j, carry):
        pltpu.sync_copy(ones_b, cnt_sh.at[idxd.at[j]], add=True)
        return carry

    lax.fori_loop(0, CH, chunk, 0)
    plsc.subcore_barrier()
    for t in range(SRS // CW):
        rows = pl.ds(sid * SRS + t * CW, CW)
        pltpu.sync_copy(cnt_sh.at[rows, :], cntp.at[cid, rows, :])


@functools.cache
def _build_count():
    return pl.kernel(
        _count_body,
        out_type=jax.ShapeDtypeStruct((NC, NP, H2), _f32),
        mesh=_mesh(),
        scratch_types=[
            pltpu.VMEM((CH, CW), jnp.int32),
            pltpu.VMEM((CW, H2), _f32),
            pltpu.VMEM((CW, H2), _f32),
            pltpu.VMEM_SHARED((NP, H2), _f32),
        ],
    )


def _count(dst3):
    return _build_count()(dst3)


def kernel(x, edge_index, edge_attr, params):
    src = edge_index[0].reshape(NW, CH, CW)
    dst = edge_index[1].reshape(NW, CH, CW)
    ea2 = edge_attr.reshape(E2, 2 * DE)

    def r1(v):
        return v.reshape(1, -1)

    pn = params['node_mlp']
    pe = params['edge_mlp']
    gnn = params['gnn']
    agg = params['agg']

    lp0 = gnn[0]
    node, t = _head(
        x, pn['W'], r1(pn['b']), r1(pn['g']), r1(pn['be']),
        lp0['mW1'][:, :H], r1(lp0['mb1']), lp0['mW1'][:, H:])

    u, st_e = _edge_u(ea2, _bdiag(pe['W']), _tile2(pe['b']))
    cntp = _count(dst)

    for li, lp in enumerate(gnn):
        s, st_s = _gather_swish(t, dst, src)
        st_s = st_s.reshape(NW, 2, H)
        msg = _msg(s, st_s, _bdiag(lp['mW2']), _tile2(lp['mb2']),
                   r1(lp['mg']), r1(lp['mbe']))
        aggp = _scatter_add(msg, dst)
        if li + 1 < len(gnn):
            nxt = gnn[li + 1]
            pwd, pbd, pws = nxt['mW1'][:, :H], r1(nxt['mb1']), nxt['mW1'][:, H:]
        else:
            aw1 = agg[0]['W']
            pwd = aw1[:, :H]
            pbd = jnp.zeros((1, H), _f32)
            pws = aw1[:, :H]
        node, t = _update(
            node, aggp, cntp, lp['uW1'][:, :H], lp['uW1'][:, H:],
            r1(lp['ub1']), lp['uW2'], r1(lp['ub2']), pwd, pbd, pws)

    d = _gather_diff(t, dst, src)
    t1, st1 = _t1(d, u, st_e, _tile2(pe['g']), _tile2(pe['be']),
                  _bdiag(agg[0]['W'][:, H:]), _tile2(agg[0]['b']))
    s2, st2 = _t2(t1, st1, _tile2(agg[0]['g']), _tile2(agg[0]['be']),
                  _bdiag(agg[1]['W']), _tile2(agg[1]['b']))
    y2 = _finout(s2, st2, _tile2(agg[1]['g']), _tile2(agg[1]['be']))
    return y2.reshape(E, H)


# speedup vs baseline: 1.9336x; 1.2152x over previous
import functools

import jax
import jax.numpy as jnp
from jax import lax
from jax.experimental import pallas as pl
from jax.experimental.pallas import tpu as pltpu
from jax.experimental.pallas import tpu_sc as plsc

N = 10000
E = 320000
E2 = E // 2
D = 128
DE = 16
H = 64
H2 = 2 * H
EPS = 1e-5

NC = 2
NS = 16
NW = NC * NS
EW = E // NW
CW = 80
CP = CW // 2
CH = EW // CW
NP = 10240
SRS = NP // NS
CE = 8000
GE = E2 // CE

_f32 = jnp.float32


def _swish(x):
    return x * (1.0 / (1.0 + jnp.exp(-x)))


def _mm_t(x, w):
    return lax.dot_general(x, w, (((1,), (1,)), ((), ())),
                           preferred_element_type=_f32)


def _bdiag(w):
    o, i = w.shape
    z = jnp.zeros((o, i), w.dtype)
    return jnp.concatenate(
        [jnp.concatenate([w, z], axis=1), jnp.concatenate([z, w], axis=1)],
        axis=0)


def _tile2(v):
    return jnp.concatenate([v, v]).reshape(1, -1)


def _bn_affine_paired(stats, g, be):
    ssum = stats[0:1, :H] + stats[0:1, H:]
    ssq = stats[1:2, :H] + stats[1:2, H:]
    m = ssum / E
    v = ssq / E - m * m
    scale = g[:, :H] / jnp.sqrt(v + EPS)
    shift = be[:, :H] - m * scale
    scale2 = jnp.concatenate([scale, scale], axis=1)
    shift2 = jnp.concatenate([shift, shift], axis=1)
    return scale2, shift2


def _head_body(x_ref, w_ref, b_ref, g_ref, be_ref, pwd_ref, pbd_ref, pws_ref,
               node_ref, t_ref):
    h = _swish(_mm_t(x_ref[...], w_ref[...]) + b_ref[...])
    m = jnp.mean(h, axis=0, keepdims=True)
    v = jnp.mean((h - m) * (h - m), axis=0, keepdims=True)
    node = g_ref[...] * (h - m) / jnp.sqrt(v + EPS) + be_ref[...]
    node_ref[...] = node
    ta = _mm_t(node, pwd_ref[...]) + pbd_ref[...]
    tb = _mm_t(node, pws_ref[...])
    t_ref[...] = jnp.concatenate([ta, tb], axis=1)


def _head(x, w, b, g, be, pwd, pbd, pws):
    out = [jax.ShapeDtypeStruct((N, H), _f32),
           jax.ShapeDtypeStruct((N, H2), _f32)]
    return pl.pallas_call(_head_body, out_shape=out)(
        x, w, b, g, be, pwd, pbd, pws)


def _edge_u_body(ea_ref, w_ref, b_ref, u_ref, st_ref, acc_ref):
    i = pl.program_id(0)
    s = _swish(_mm_t(ea_ref[...], w_ref[...]) + b_ref[...])
    u_ref[...] = s

    @pl.when(i == 0)
    def _():
        acc_ref[...] = jnp.zeros_like(acc_ref)

    acc_ref[0:1, :] += jnp.sum(s, axis=0, keepdims=True)
    acc_ref[1:2, :] += jnp.sum(s * s, axis=0, keepdims=True)

    @pl.when(i == pl.num_programs(0) - 1)
    def _():
        st_ref[...] = acc_ref[...]


def _edge_u(ea2, w2, b2):
    return pl.pallas_call(
        _edge_u_body,
        grid=(GE,),
        in_specs=[
            pl.BlockSpec((CE, 2 * DE), lambda i: (i, 0)),
            pl.BlockSpec((H2, 2 * DE), lambda i: (0, 0)),
            pl.BlockSpec((1, H2), lambda i: (0, 0)),
        ],
        out_specs=[
            pl.BlockSpec((CE, H2), lambda i: (i, 0)),
            pl.BlockSpec((2, H2), lambda i: (0, 0)),
        ],
        out_shape=[
            jax.ShapeDtypeStruct((E2, H2), _f32),
            jax.ShapeDtypeStruct((2, H2), _f32),
        ],
        scratch_shapes=[pltpu.VMEM((2, H2), _f32)],
    )(ea2, w2, b2)


def _msg_body(s_ref, st_ref, w_ref, b_ref, g_ref, be_ref, out_ref):
    stats = jnp.sum(st_ref[...], axis=0)
    m = stats[0:1, :] / E
    v = stats[1:2, :] / E - m * m
    scale = g_ref[...] / jnp.sqrt(v + EPS)
    shift = be_ref[...] - m * scale
    scale2 = jnp.concatenate([scale, scale], axis=1)
    shift2 = jnp.concatenate([shift, shift], axis=1)
    sn = s_ref[...] * scale2 + shift2
    out_ref[...] = _swish(_mm_t(sn, w_ref[...]) + b_ref[...])


def _msg(s, st, w2, b2, g, be):
    return pl.pallas_call(
        _msg_body,
        grid=(GE,),
        in_specs=[
            pl.BlockSpec((CE, H2), lambda i: (i, 0)),
            pl.BlockSpec((NW, 2, H), lambda i: (0, 0, 0)),
            pl.BlockSpec((H2, H2), lambda i: (0, 0)),
            pl.BlockSpec((1, H2), lambda i: (0, 0)),
            pl.BlockSpec((1, H), lambda i: (0, 0)),
            pl.BlockSpec((1, H), lambda i: (0, 0)),
        ],
        out_specs=pl.BlockSpec((CE, H2), lambda i: (i, 0)),
        out_shape=jax.ShapeDtypeStruct((E2, H2), _f32),
    )(s, st, w2, b2, g, be)


def _update_body(node_ref, aggp_ref, cnt_ref, uw1a_ref, uw1b_ref, ub1_ref,
                 uw2_ref, ub2_ref, pwd_ref, pbd_ref, pws_ref,
                 nn_ref, t_ref):
    node = node_ref[...]
    agg = aggp_ref[0, :N, :H] + aggp_ref[1, :N, :H]
    deg = cnt_ref[0, :N, 0:1] + cnt_ref[1, :N, 0:1]
    mean = agg / jnp.maximum(deg, 1.0)
    upd = _swish(_mm_t(node, uw1a_ref[...]) + _mm_t(mean, uw1b_ref[...])
                 + ub1_ref[...])
    upd = _swish(_mm_t(upd, uw2_ref[...]) + ub2_ref[...])
    nn = node + upd
    nn_ref[...] = nn
    ta = _mm_t(nn, pwd_ref[...]) + pbd_ref[...]
    tb = _mm_t(nn, pws_ref[...])
    t_ref[...] = jnp.concatenate([ta, tb], axis=1)


def _update(node, aggp, cnt, uw1a, uw1b, ub1, uw2, ub2, pwd, pbd, pws):
    out = [jax.ShapeDtypeStruct((N, H), _f32),
           jax.ShapeDtypeStruct((N, H2), _f32)]
    return pl.pallas_call(_update_body, out_shape=out)(
        node, aggp, cnt, uw1a, uw1b, ub1, uw2, ub2, pwd, pbd, pws)


def _t1_body(d_ref, u_ref, ste_ref, eg_ref, ebe_ref, w_ref, b_ref,
             t1_ref, st_ref, acc_ref):
    i = pl.program_id(0)
    scale2, shift2 = _bn_affine_paired(ste_ref[...], eg_ref[...], ebe_ref[...])
    edge = u_ref[...] * scale2 + shift2
    s = _swish(d_ref[...] + _mm_t(edge, w_ref[...]) + b_ref[...])
    t1_ref[...] = s

    @pl.when(i == 0)
    def _():
        acc_ref[...] = jnp.zeros_like(acc_ref)

    acc_ref[0:1, :] += jnp.sum(s, axis=0, keepdims=True)
    acc_ref[1:2, :] += jnp.sum(s * s, axis=0, keepdims=True)

    @pl.when(i == pl.num_programs(0) - 1)
    def _():
        st_ref[...] = acc_ref[...]


def _t1(d, u, ste, eg2, ebe2, w2, b2):
    return pl.pallas_call(
        _t1_body,
        grid=(GE,),
        in_specs=[
            pl.BlockSpec((CE, H2), lambda i: (i, 0)),
            pl.BlockSpec((CE, H2), lambda i: (i, 0)),
            pl.BlockSpec((2, H2), lambda i: (0, 0)),
            pl.BlockSpec((1, H2), lambda i: (0, 0)),
            pl.BlockSpec((1, H2), lambda i: (0, 0)),
            pl.BlockSpec((H2, H2), lambda i: (0, 0)),
            pl.BlockSpec((1, H2), lambda i: (0, 0)),
        ],
        out_specs=[
            pl.BlockSpec((CE, H2), lambda i: (i, 0)),
            pl.BlockSpec((2, H2), lambda i: (0, 0)),
        ],
        out_shape=[
            jax.ShapeDtypeStruct((E2, H2), _f32),
            jax.ShapeDtypeStruct((2, H2), _f32),
        ],
        scratch_shapes=[pltpu.VMEM((2, H2), _f32)],
    )(d, u, ste, eg2, ebe2, w2, b2)


def _t2_body(t1_ref, st1_ref, g1_ref, be1_ref, w_ref, b_ref,
             s2_ref, st_ref, acc_ref):
    i = pl.program_id(0)
    scale2, shift2 = _bn_affine_paired(st1_ref[...], g1_ref[...], be1_ref[...])
    tn = t1_ref[...] * scale2 + shift2
    s = _swish(_mm_t(tn, w_ref[...]) + b_ref[...])
    s2_ref[...] = s

    @pl.when(i == 0)
    def _():
        acc_ref[...] = jnp.zeros_like(acc_ref)

    acc_ref[0:1, :] += jnp.sum(s, axis=0, keepdims=True)
    acc_ref[1:2, :] += jnp.sum(s * s, axis=0, keepdims=True)

    @pl.when(i == pl.num_programs(0) - 1)
    def _():
        st_ref[...] = acc_ref[...]


def _t2(t1, st1, g12, be12, w2, b2):
    return pl.pallas_call(
        _t2_body,
        grid=(GE,),
        in_specs=[
            pl.BlockSpec((CE, H2), lambda i: (i, 0)),
            pl.BlockSpec((2, H2), lambda i: (0, 0)),
            pl.BlockSpec((1, H2), lambda i: (0, 0)),
            pl.BlockSpec((1, H2), lambda i: (0, 0)),
            pl.BlockSpec((H2, H2), lambda i: (0, 0)),
            pl.BlockSpec((1, H2), lambda i: (0, 0)),
        ],
        out_specs=[
            pl.BlockSpec((CE, H2), lambda i: (i, 0)),
            pl.BlockSpec((2, H2), lambda i: (0, 0)),
        ],
        out_shape=[
            jax.ShapeDtypeStruct((E2, H2), _f32),
            jax.ShapeDtypeStruct((2, H2), _f32),
        ],
        scratch_shapes=[pltpu.VMEM((2, H2), _f32)],
    )(t1, st1, g12, be12, w2, b2)


def _finout_body(s2_ref, st2_ref, g2_ref, be2_ref, y_ref):
    scale2, shift2 = _bn_affine_paired(st2_ref[...], g2_ref[...], be2_ref[...])
    y_ref[...] = s2_ref[...] * scale2 + shift2


def _finout(s2, st2, g22, be22):
    return pl.pallas_call(
        _finout_body,
        grid=(GE,),
        in_specs=[
            pl.BlockSpec((CE, H2), lambda i: (i, 0)),
            pl.BlockSpec((2, H2), lambda i: (0, 0)),
            pl.BlockSpec((1, H2), lambda i: (0, 0)),
            pl.BlockSpec((1, H2), lambda i: (0, 0)),
        ],
        out_specs=pl.BlockSpec((CE, H2), lambda i: (i, 0)),
        out_shape=jax.ShapeDtypeStruct((E2, H2), _f32),
    )(s2, st2, g22, be22)


@functools.cache
def _mesh():
    return plsc.VectorSubcoreMesh(core_axis_name="c", subcore_axis_name="s",
                                  num_cores=NC, num_subcores=NS)


def _worker_id():
    return lax.axis_index("s") * NC + lax.axis_index("c")


def _make_gather_body(with_stats):
    def body(t_hbm, dst_hbm, src_hbm, s_hbm, *rest):
        if with_stats:
            (st_hbm, idxd, idxs, ra0, rb0, ra1, rb1, so0, so1,
             sg0, sg1, semo) = rest
        else:
            (idxd, idxs, ra0, rb0, ra1, rb1, so0, so1,
             sg0, sg1, semo) = rest
        wid = _worker_id()
        obase = wid * (EW // 2)
        pltpu.sync_copy(dst_hbm.at[wid], idxd)
        pltpu.sync_copy(src_hbm.at[wid], idxs)

        def fire(j, ra, rb, sg):
            pltpu.async_copy(t_hbm.at[idxd.at[j]], ra, sg)
            pltpu.async_copy(t_hbm.at[idxs.at[j]], rb, sg)

        def drain_g(ra, rb, sg):
            pltpu.make_async_copy(t_hbm.at[pl.ds(0, CW)], ra, sg).wait()
            pltpu.make_async_copy(t_hbm.at[pl.ds(0, CW)], rb, sg).wait()

        def drain_o(so):
            pltpu.make_async_copy(so, s_hbm.at[pl.ds(0, CP), :], semo).wait()

        def compute2(ra, rb, so, stats):
            def pair(p, c):
                c = list(c) if with_stats else c
                for half in range(2):
                    e = 2 * p + half
                    for q in range(4):
                        a = ra[e, pl.ds(q * 16, 16)]
                        b = rb[e, pl.ds(H + q * 16, 16)]
                        if with_stats:
                            h = a + b
                            v = h / (1.0 + jnp.exp(-h))
                            c[q] = c[q] + v
                            c[q + 4] = c[q + 4] + v * v
                        else:
                            v = a - b
                        so[p, pl.ds(half * H + q * 16, 16)] = v
                return tuple(c) if with_stats else c
            return lax.fori_loop(0, CP, pair, stats)

        zz = jnp.zeros((16,), _f32)
        stats = tuple(zz for _ in range(8)) if with_stats else 0

        fire(0, ra0, rb0, sg0)

        def step(k, stats):
            j0 = 2 * k
            fire(j0 + 1, ra1, rb1, sg1)
            drain_g(ra0, rb0, sg0)

            @pl.when(j0 >= 2)
            def _():
                drain_o(so0)

            stats = compute2(ra0, rb0, so0, stats)
            pltpu.async_copy(so0, s_hbm.at[pl.ds(obase + j0 * CP, CP), :],
                             semo)
            jn = j0 + 2

            @pl.when(jn < CH)
            def _():
                fire(jn, ra0, rb0, sg0)

            drain_g(ra1, rb1, sg1)

            @pl.when(j0 + 1 >= 2)
            def _():
                drain_o(so1)

            stats = compute2(ra1, rb1, so1, stats)
            pltpu.async_copy(so1, s_hbm.at[pl.ds(obase + (j0 + 1) * CP, CP),
                                           :], semo)
            return stats

        stats = lax.fori_loop(0, CH // 2, step, stats)
        jl = CH - 1
        drain_g(ra0, rb0, sg0)
        drain_o(so0)
        stats = compute2(ra0, rb0, so0, stats)
        pltpu.async_copy(so0, s_hbm.at[pl.ds(obase + jl * CP, CP), :], semo)
        drain_o(so1)
        drain_o(so0)

        if with_stats:
            stv = so1
            for q in range(8):
                stv[0, pl.ds(q * 16, 16)] = stats[q]
            pltpu.sync_copy(stv.at[pl.ds(0, 1), :], st_hbm.at[wid])

    return body


@functools.cache
def _build_gather_swish():
    return pl.kernel(
        _make_gather_body(True),
        out_type=[
            jax.ShapeDtypeStruct((E2, H2), _f32),
            jax.ShapeDtypeStruct((NW, 1, H2), _f32),
        ],
        mesh=_mesh(),
        scratch_types=[
            pltpu.VMEM((CH, CW), jnp.int32),
            pltpu.VMEM((CH, CW), jnp.int32),
            pltpu.VMEM((CW, H2), _f32),
            pltpu.VMEM((CW, H2), _f32),
            pltpu.VMEM((CW, H2), _f32),
            pltpu.VMEM((CW, H2), _f32),
            pltpu.VMEM((CP, H2), _f32),
            pltpu.VMEM((CP, H2), _f32),
            pltpu.SemaphoreType.DMA,
            pltpu.SemaphoreType.DMA,
            pltpu.SemaphoreType.DMA,
        ],
    )


def _gather_swish(t, dst3, src3):
    return _build_gather_swish()(t, dst3, src3)


@functools.cache
def _build_gather_diff():
    return pl.kernel(
        _make_gather_body(False),
        out_type=jax.ShapeDtypeStruct((E2, H2), _f32),
        mesh=_mesh(),
        scratch_types=[
            pltpu.VMEM((CH, CW), jnp.int32),
            pltpu.VMEM((CH, CW), jnp.int32),
            pltpu.VMEM((CW, H2), _f32),
            pltpu.VMEM((CW, H2), _f32),
            pltpu.VMEM((CW, H2), _f32),
            pltpu.VMEM((CW, H2), _f32),
            pltpu.VMEM((CP, H2), _f32),
            pltpu.VMEM((CP, H2), _f32),
            pltpu.SemaphoreType.DMA,
            pltpu.SemaphoreType.DMA,
            pltpu.SemaphoreType.DMA,
        ],
    )


def _gather_diff(t, dst3, src3):
    return _build_gather_diff()(t, dst3, src3)


def _scatter_add_body(msg, dst3, aggp, idxd, mb, mb2, agg_sh):
    cid = lax.axis_index("c")
    sid = lax.axis_index("s")
    wid = sid * NC + cid
    pltpu.sync_copy(dst3.at[wid], idxd)

    zz = jnp.zeros((16,), _f32)

    def zrow(r, c):
        for q in range(8):
            mb2[r, pl.ds(q * 16, 16)] = zz
        return c

    lax.fori_loop(0, CW, zrow, 0)
    for t in range(SRS // CW):
        pltpu.sync_copy(mb2, agg_sh.at[pl.ds(sid * SRS + t * CW, CW), :])
    plsc.subcore_barrier()

    def chunk(j, carry):
        pltpu.sync_copy(msg.at[pl.ds(wid * (EW // 2) + j * CP, CP), :], mb)

        def pair(p, c2):
            for q in range(4):
                mb2[2 * p, pl.ds(q * 16, 16)] = mb[p, pl.ds(q * 16, 16)]
                mb2[2 * p + 1, pl.ds(q * 16, 16)] = mb[p, pl.ds(H + q * 16, 16)]
            return c2

        lax.fori_loop(0, CP, pair, 0)
        pltpu.sync_copy(mb2, agg_sh.at[idxd.at[j]], add=True)
        return carry

    lax.fori_loop(0, CH, chunk, 0)
    plsc.subcore_barrier()
    for t in range(SRS // CW):
        rows = pl.ds(sid * SRS + t * CW, CW)
        pltpu.sync_copy(agg_sh.at[rows, :], aggp.at[cid, rows, :])


@functools.cache
def _build_scatter_add():
    return pl.kernel(
        _scatter_add_body,
        out_type=jax.ShapeDtypeStruct((NC, NP, H2), _f32),
        mesh=_mesh(),
        scratch_types=[
            pltpu.VMEM((CH, CW), jnp.int32),
            pltpu.VMEM((CP, H2), _f32),
            pltpu.VMEM((CW, H2), _f32),
            pltpu.VMEM_SHARED((NP, H2), _f32),
        ],
    )


def _scatter_add(msg, dst3):
    return _build_scatter_add()(msg, dst3)


def _count_body(dst3, cntp, idxd, ones_b, zb, cnt_sh):
    cid = lax.axis_index("c")
    sid = lax.axis_index("s")
    wid = sid * NC + cid
    pltpu.sync_copy(dst3.at[wid], idxd)

    zz = jnp.zeros((16,), _f32)
    oo = jnp.ones((16,), _f32)

    def frow(r, c):
        for q in range(8):
            zb[r, pl.ds(q * 16, 16)] = zz
            ones_b[r, pl.ds(q * 16, 16)] = oo
        return c

    lax.fori_loop(0, CW, frow, 0)
    for t in range(SRS // CW):
        pltpu.sync_copy(zb, cnt_sh.at[pl.ds(sid * SRS + t * CW, CW), :])
    plsc.subcore_barrier()

    def chunk(j, carry):
        pltpu.sync_copy(ones_b, cnt_sh.at[idxd.at[j]], add=True)
        return carry

    lax.fori_loop(0, CH, chunk, 0)
    plsc.subcore_barrier()
    for t in range(SRS // CW):
        rows = pl.ds(sid * SRS + t * CW, CW)
        pltpu.sync_copy(cnt_sh.at[rows, :], cntp.at[cid, rows, :])


@functools.cache
def _build_count():
    return pl.kernel(
        _count_body,
        out_type=jax.ShapeDtypeStruct((NC, NP, H2), _f32),
        mesh=_mesh(),
        scratch_types=[
            pltpu.VMEM((CH, CW), jnp.int32),
            pltpu.VMEM((CW, H2), _f32),
            pltpu.VMEM((CW, H2), _f32),
            pltpu.VMEM_SHARED((NP, H2), _f32),
        ],
    )


def _count(dst3):
    return _build_count()(dst3)


def kernel(x, edge_index, edge_attr, params):
    src = edge_index[0].reshape(NW, CH, CW)
    dst = edge_index[1].reshape(NW, CH, CW)
    ea2 = edge_attr.reshape(E2, 2 * DE)

    def r1(v):
        return v.reshape(1, -1)

    pn = params['node_mlp']
    pe = params['edge_mlp']
    gnn = params['gnn']
    agg = params['agg']

    lp0 = gnn[0]
    node, t = _head(
        x, pn['W'], r1(pn['b']), r1(pn['g']), r1(pn['be']),
        lp0['mW1'][:, :H], r1(lp0['mb1']), lp0['mW1'][:, H:])

    u, st_e = _edge_u(ea2, _bdiag(pe['W']), _tile2(pe['b']))
    cntp = _count(dst)

    for li, lp in enumerate(gnn):
        s, st_s = _gather_swish(t, dst, src)
        st_s = st_s.reshape(NW, 2, H)
        msg = _msg(s, st_s, _bdiag(lp['mW2']), _tile2(lp['mb2']),
                   r1(lp['mg']), r1(lp['mbe']))
        aggp = _scatter_add(msg, dst)
        if li + 1 < len(gnn):
            nxt = gnn[li + 1]
            pwd, pbd, pws = nxt['mW1'][:, :H], r1(nxt['mb1']), nxt['mW1'][:, H:]
        else:
            aw1 = agg[0]['W']
            pwd = aw1[:, :H]
            pbd = jnp.zeros((1, H), _f32)
            pws = aw1[:, :H]
        node, t = _update(
            node, aggp, cntp, lp['uW1'][:, :H], lp['uW1'][:, H:],
            r1(lp['ub1']), lp['uW2'], r1(lp['ub2']), pwd, pbd, pws)

    d = _gather_diff(t, dst, src)
    t1, st1 = _t1(d, u, st_e, _tile2(pe['g']), _tile2(pe['be']),
                  _bdiag(agg[0]['W'][:, H:]), _tile2(agg[0]['b']))
    s2, st2 = _t2(t1, st1, _tile2(agg[0]['g']), _tile2(agg[0]['be']),
                  _bdiag(agg[1]['W']), _tile2(agg[1]['b']))
    y2 = _finout(s2, st2, _tile2(agg[1]['g']), _tile2(agg[1]['be']))
    return y2.reshape(E, H)


# speedup vs baseline: 3.1569x; 1.6326x over previous
import functools

import jax
import jax.numpy as jnp
from jax import lax
from jax.experimental import pallas as pl
from jax.experimental.pallas import tpu as pltpu
from jax.experimental.pallas import tpu_sc as plsc

N = 10000
E = 320000
E2 = E // 2
D = 128
DE = 16
H = 64
H2 = 2 * H
EPS = 1e-5

NC = 2
NS = 16
NW = NC * NS
EW = E // NW
CW = 80
CP = CW // 2
CH = EW // CW
NP = 10240
SRS = NP // NS
CE = 8000
GE = E2 // CE

_f32 = jnp.float32


def _swish(x):
    return x * (1.0 / (1.0 + jnp.exp(-x)))


def _mm_t(x, w):
    return lax.dot_general(x, w, (((1,), (1,)), ((), ())),
                           preferred_element_type=_f32)


def _bdiag(w):
    o, i = w.shape
    z = jnp.zeros((o, i), w.dtype)
    return jnp.concatenate(
        [jnp.concatenate([w, z], axis=1), jnp.concatenate([z, w], axis=1)],
        axis=0)


def _tile2(v):
    return jnp.concatenate([v, v]).reshape(1, -1)


def _bn_affine_paired(stats, g, be):
    ssum = stats[0:1, :H] + stats[0:1, H:]
    ssq = stats[1:2, :H] + stats[1:2, H:]
    m = ssum / E
    v = ssq / E - m * m
    scale = g[:, :H] / jnp.sqrt(v + EPS)
    shift = be[:, :H] - m * scale
    scale2 = jnp.concatenate([scale, scale], axis=1)
    shift2 = jnp.concatenate([shift, shift], axis=1)
    return scale2, shift2


def _head_body(x_ref, w_ref, b_ref, g_ref, be_ref, pwd_ref, pbd_ref, pws_ref,
               node_ref, t_ref):
    h = _swish(_mm_t(x_ref[...], w_ref[...]) + b_ref[...])
    m = jnp.mean(h, axis=0, keepdims=True)
    v = jnp.mean((h - m) * (h - m), axis=0, keepdims=True)
    node = g_ref[...] * (h - m) / jnp.sqrt(v + EPS) + be_ref[...]
    node_ref[...] = node
    ta = _mm_t(node, pwd_ref[...]) + pbd_ref[...]
    tb = _mm_t(node, pws_ref[...])
    t_ref[...] = jnp.concatenate([ta, tb], axis=1)


def _head(x, w, b, g, be, pwd, pbd, pws):
    out = [jax.ShapeDtypeStruct((N, H), _f32),
           jax.ShapeDtypeStruct((N, H2), _f32)]
    return pl.pallas_call(_head_body, out_shape=out)(
        x, w, b, g, be, pwd, pbd, pws)


def _edge_u_body(ea_ref, w_ref, b_ref, u_ref, st_ref, acc_ref):
    i = pl.program_id(0)
    s = _swish(_mm_t(ea_ref[...], w_ref[...]) + b_ref[...])
    u_ref[...] = s

    @pl.when(i == 0)
    def _():
        acc_ref[...] = jnp.zeros_like(acc_ref)

    acc_ref[0:1, :] += jnp.sum(s, axis=0, keepdims=True)
    acc_ref[1:2, :] += jnp.sum(s * s, axis=0, keepdims=True)

    @pl.when(i == pl.num_programs(0) - 1)
    def _():
        st_ref[...] = acc_ref[...]


def _edge_u(ea2, w2, b2):
    return pl.pallas_call(
        _edge_u_body,
        grid=(GE,),
        in_specs=[
            pl.BlockSpec((CE, 2 * DE), lambda i: (i, 0)),
            pl.BlockSpec((H2, 2 * DE), lambda i: (0, 0)),
            pl.BlockSpec((1, H2), lambda i: (0, 0)),
        ],
        out_specs=[
            pl.BlockSpec((CE, H2), lambda i: (i, 0)),
            pl.BlockSpec((2, H2), lambda i: (0, 0)),
        ],
        out_shape=[
            jax.ShapeDtypeStruct((E2, H2), _f32),
            jax.ShapeDtypeStruct((2, H2), _f32),
        ],
        scratch_shapes=[pltpu.VMEM((2, H2), _f32)],
    )(ea2, w2, b2)


def _hstats_body(h_ref, st_ref, acc_ref):
    i = pl.program_id(0)
    s = _swish(h_ref[...])

    @pl.when(i == 0)
    def _():
        acc_ref[...] = jnp.zeros_like(acc_ref)

    acc_ref[0:1, :] += jnp.sum(s, axis=0, keepdims=True)
    acc_ref[1:2, :] += jnp.sum(s * s, axis=0, keepdims=True)

    @pl.when(i == pl.num_programs(0) - 1)
    def _():
        st_ref[...] = acc_ref[...]


def _hstats(h):
    return pl.pallas_call(
        _hstats_body,
        grid=(GE,),
        in_specs=[pl.BlockSpec((CE, H2), lambda i: (i, 0))],
        out_specs=pl.BlockSpec((2, H2), lambda i: (0, 0)),
        out_shape=jax.ShapeDtypeStruct((2, H2), _f32),
        scratch_shapes=[pltpu.VMEM((2, H2), _f32)],
    )(h)


def _msg_body(h_ref, st_ref, w_ref, b_ref, g_ref, be_ref, out_ref):
    scale2, shift2 = _bn_affine_paired(st_ref[...], g_ref[...], be_ref[...])
    sn = _swish(h_ref[...]) * scale2 + shift2
    out_ref[...] = _swish(_mm_t(sn, w_ref[...]) + b_ref[...])


def _msg(h, st, w2, b2, g2, be2):
    return pl.pallas_call(
        _msg_body,
        grid=(GE,),
        in_specs=[
            pl.BlockSpec((CE, H2), lambda i: (i, 0)),
            pl.BlockSpec((2, H2), lambda i: (0, 0)),
            pl.BlockSpec((H2, H2), lambda i: (0, 0)),
            pl.BlockSpec((1, H2), lambda i: (0, 0)),
            pl.BlockSpec((1, H2), lambda i: (0, 0)),
            pl.BlockSpec((1, H2), lambda i: (0, 0)),
        ],
        out_specs=pl.BlockSpec((CE, H2), lambda i: (i, 0)),
        out_shape=jax.ShapeDtypeStruct((E2, H2), _f32),
    )(h, st, w2, b2, g2, be2)


def _update_body(node_ref, aggp_ref, cnt_ref, uw1a_ref, uw1b_ref, ub1_ref,
                 uw2_ref, ub2_ref, pwd_ref, pbd_ref, pws_ref,
                 nn_ref, t_ref):
    node = node_ref[...]
    agg = aggp_ref[0, :N, :H] + aggp_ref[1, :N, :H]
    deg = cnt_ref[0, :N, 0:1] + cnt_ref[1, :N, 0:1]
    mean = agg / jnp.maximum(deg, 1.0)
    upd = _swish(_mm_t(node, uw1a_ref[...]) + _mm_t(mean, uw1b_ref[...])
                 + ub1_ref[...])
    upd = _swish(_mm_t(upd, uw2_ref[...]) + ub2_ref[...])
    nn = node + upd
    nn_ref[...] = nn
    ta = _mm_t(nn, pwd_ref[...]) + pbd_ref[...]
    tb = _mm_t(nn, pws_ref[...])
    t_ref[...] = jnp.concatenate([ta, tb], axis=1)


def _update(node, aggp, cnt, uw1a, uw1b, ub1, uw2, ub2, pwd, pbd, pws):
    out = [jax.ShapeDtypeStruct((N, H), _f32),
           jax.ShapeDtypeStruct((N, H2), _f32)]
    return pl.pallas_call(_update_body, out_shape=out)(
        node, aggp, cnt, uw1a, uw1b, ub1, uw2, ub2, pwd, pbd, pws)


def _t1_body(d_ref, u_ref, ste_ref, eg_ref, ebe_ref, w_ref, b_ref,
             t1_ref, st_ref, acc_ref):
    i = pl.program_id(0)
    scale2, shift2 = _bn_affine_paired(ste_ref[...], eg_ref[...], ebe_ref[...])
    edge = u_ref[...] * scale2 + shift2
    s = _swish(d_ref[...] + _mm_t(edge, w_ref[...]) + b_ref[...])
    t1_ref[...] = s

    @pl.when(i == 0)
    def _():
        acc_ref[...] = jnp.zeros_like(acc_ref)

    acc_ref[0:1, :] += jnp.sum(s, axis=0, keepdims=True)
    acc_ref[1:2, :] += jnp.sum(s * s, axis=0, keepdims=True)

    @pl.when(i == pl.num_programs(0) - 1)
    def _():
        st_ref[...] = acc_ref[...]


def _t1(d, u, ste, eg2, ebe2, w2, b2):
    return pl.pallas_call(
        _t1_body,
        grid=(GE,),
        in_specs=[
            pl.BlockSpec((CE, H2), lambda i: (i, 0)),
            pl.BlockSpec((CE, H2), lambda i: (i, 0)),
            pl.BlockSpec((2, H2), lambda i: (0, 0)),
            pl.BlockSpec((1, H2), lambda i: (0, 0)),
            pl.BlockSpec((1, H2), lambda i: (0, 0)),
            pl.BlockSpec((H2, H2), lambda i: (0, 0)),
            pl.BlockSpec((1, H2), lambda i: (0, 0)),
        ],
        out_specs=[
            pl.BlockSpec((CE, H2), lambda i: (i, 0)),
            pl.BlockSpec((2, H2), lambda i: (0, 0)),
        ],
        out_shape=[
            jax.ShapeDtypeStruct((E2, H2), _f32),
            jax.ShapeDtypeStruct((2, H2), _f32),
        ],
        scratch_shapes=[pltpu.VMEM((2, H2), _f32)],
    )(d, u, ste, eg2, ebe2, w2, b2)


def _t2_body(t1_ref, st1_ref, g1_ref, be1_ref, w_ref, b_ref,
             s2_ref, st_ref, acc_ref):
    i = pl.program_id(0)
    scale2, shift2 = _bn_affine_paired(st1_ref[...], g1_ref[...], be1_ref[...])
    tn = t1_ref[...] * scale2 + shift2
    s = _swish(_mm_t(tn, w_ref[...]) + b_ref[...])
    s2_ref[...] = s

    @pl.when(i == 0)
    def _():
        acc_ref[...] = jnp.zeros_like(acc_ref)

    acc_ref[0:1, :] += jnp.sum(s, axis=0, keepdims=True)
    acc_ref[1:2, :] += jnp.sum(s * s, axis=0, keepdims=True)

    @pl.when(i == pl.num_programs(0) - 1)
    def _():
        st_ref[...] = acc_ref[...]


def _t2(t1, st1, g12, be12, w2, b2):
    return pl.pallas_call(
        _t2_body,
        grid=(GE,),
        in_specs=[
            pl.BlockSpec((CE, H2), lambda i: (i, 0)),
            pl.BlockSpec((2, H2), lambda i: (0, 0)),
            pl.BlockSpec((1, H2), lambda i: (0, 0)),
            pl.BlockSpec((1, H2), lambda i: (0, 0)),
            pl.BlockSpec((H2, H2), lambda i: (0, 0)),
            pl.BlockSpec((1, H2), lambda i: (0, 0)),
        ],
        out_specs=[
            pl.BlockSpec((CE, H2), lambda i: (i, 0)),
            pl.BlockSpec((2, H2), lambda i: (0, 0)),
        ],
        out_shape=[
            jax.ShapeDtypeStruct((E2, H2), _f32),
            jax.ShapeDtypeStruct((2, H2), _f32),
        ],
        scratch_shapes=[pltpu.VMEM((2, H2), _f32)],
    )(t1, st1, g12, be12, w2, b2)


def _finout_body(s2_ref, st2_ref, g2_ref, be2_ref, y_ref):
    scale2, shift2 = _bn_affine_paired(st2_ref[...], g2_ref[...], be2_ref[...])
    y_ref[...] = s2_ref[...] * scale2 + shift2


def _finout(s2, st2, g22, be22):
    return pl.pallas_call(
        _finout_body,
        grid=(GE,),
        in_specs=[
            pl.BlockSpec((CE, H2), lambda i: (i, 0)),
            pl.BlockSpec((2, H2), lambda i: (0, 0)),
            pl.BlockSpec((1, H2), lambda i: (0, 0)),
            pl.BlockSpec((1, H2), lambda i: (0, 0)),
        ],
        out_specs=pl.BlockSpec((CE, H2), lambda i: (i, 0)),
        out_shape=jax.ShapeDtypeStruct((E2, H2), _f32),
    )(s2, st2, g22, be22)


@functools.cache
def _mesh():
    return plsc.VectorSubcoreMesh(core_axis_name="c", subcore_axis_name="s",
                                  num_cores=NC, num_subcores=NS)


def _worker_id():
    return lax.axis_index("s") * NC + lax.axis_index("c")


def _make_gather_body(diff):
    def body(t_hbm, dst_hbm, src_hbm, s_hbm,
             idxd, idxs, ra0, rb0, ra1, rb1, so0, so1, sg0, sg1, semo):
        wid = _worker_id()
        obase = wid * (EW // 2)
        pltpu.sync_copy(dst_hbm.at[wid], idxd)
        pltpu.sync_copy(src_hbm.at[wid], idxs)

        def fire(j, ra, rb, sg):
            pltpu.async_copy(t_hbm.at[idxd.at[j]], ra, sg)
            pltpu.async_copy(t_hbm.at[idxs.at[j]], rb, sg)

        def drain_g(ra, rb, sg):
            pltpu.make_async_copy(t_hbm.at[pl.ds(0, CW)], ra, sg).wait()
            pltpu.make_async_copy(t_hbm.at[pl.ds(0, CW)], rb, sg).wait()

        def drain_o(so):
            pltpu.make_async_copy(so, s_hbm.at[pl.ds(0, CP), :], semo).wait()

        def compute2(ra, rb, so):
            def pair(p, c):
                for half in range(2):
                    e = 2 * p + half
                    for q in range(4):
                        a = ra[e, pl.ds(q * 16, 16)]
                        b = rb[e, pl.ds(H + q * 16, 16)]
                        v = (a - b) if diff else (a + b)
                        so[p, pl.ds(half * H + q * 16, 16)] = v
                return c
            lax.fori_loop(0, CP, pair, 0)

        fire(0, ra0, rb0, sg0)

        def step(k, carry):
            j0 = 2 * k
            fire(j0 + 1, ra1, rb1, sg1)
            drain_g(ra0, rb0, sg0)

            @pl.when(j0 >= 2)
            def _():
                drain_o(so0)

            compute2(ra0, rb0, so0)
            pltpu.async_copy(so0, s_hbm.at[pl.ds(obase + j0 * CP, CP), :],
                             semo)
            jn = j0 + 2

            @pl.when(jn < CH)
            def _():
                fire(jn, ra0, rb0, sg0)

            drain_g(ra1, rb1, sg1)

            @pl.when(j0 + 1 >= 2)
            def _():
                drain_o(so1)

            compute2(ra1, rb1, so1)
            pltpu.async_copy(so1, s_hbm.at[pl.ds(obase + (j0 + 1) * CP, CP),
                                           :], semo)
            return carry

        lax.fori_loop(0, CH // 2, step, 0)
        jl = CH - 1
        drain_g(ra0, rb0, sg0)
        drain_o(so0)
        compute2(ra0, rb0, so0)
        pltpu.async_copy(so0, s_hbm.at[pl.ds(obase + jl * CP, CP), :], semo)
        drain_o(so1)
        drain_o(so0)

    return body


@functools.cache
def _build_gather_add():
    return pl.kernel(
        _make_gather_body(False),
        out_type=jax.ShapeDtypeStruct((E2, H2), _f32),
        mesh=_mesh(),
        scratch_types=[
            pltpu.VMEM((CH, CW), jnp.int32),
            pltpu.VMEM((CH, CW), jnp.int32),
            pltpu.VMEM((CW, H2), _f32),
            pltpu.VMEM((CW, H2), _f32),
            pltpu.VMEM((CW, H2), _f32),
            pltpu.VMEM((CW, H2), _f32),
            pltpu.VMEM((CP, H2), _f32),
            pltpu.VMEM((CP, H2), _f32),
            pltpu.SemaphoreType.DMA,
            pltpu.SemaphoreType.DMA,
            pltpu.SemaphoreType.DMA,
        ],
    )


def _gather_add(t, dst3, src3):
    return _build_gather_add()(t, dst3, src3)


@functools.cache
def _build_gather_diff():
    return pl.kernel(
        _make_gather_body(True),
        out_type=jax.ShapeDtypeStruct((E2, H2), _f32),
        mesh=_mesh(),
        scratch_types=[
            pltpu.VMEM((CH, CW), jnp.int32),
            pltpu.VMEM((CH, CW), jnp.int32),
            pltpu.VMEM((CW, H2), _f32),
            pltpu.VMEM((CW, H2), _f32),
            pltpu.VMEM((CW, H2), _f32),
            pltpu.VMEM((CW, H2), _f32),
            pltpu.VMEM((CP, H2), _f32),
            pltpu.VMEM((CP, H2), _f32),
            pltpu.SemaphoreType.DMA,
            pltpu.SemaphoreType.DMA,
            pltpu.SemaphoreType.DMA,
        ],
    )


def _gather_diff(t, dst3, src3):
    return _build_gather_diff()(t, dst3, src3)


def _scatter_add_body(msg, dst3, aggp, idxd, mb, mb2, agg_sh):
    cid = lax.axis_index("c")
    sid = lax.axis_index("s")
    wid = sid * NC + cid
    pltpu.sync_copy(dst3.at[wid], idxd)

    zz = jnp.zeros((16,), _f32)

    def zrow(r, c):
        for q in range(8):
            mb2[r, pl.ds(q * 16, 16)] = zz
        return c

    lax.fori_loop(0, CW, zrow, 0)
    for t in range(SRS // CW):
        pltpu.sync_copy(mb2, agg_sh.at[pl.ds(sid * SRS + t * CW, CW), :])
    plsc.subcore_barrier()

    def chunk(j, carry):
        pltpu.sync_copy(msg.at[pl.ds(wid * (EW // 2) + j * CP, CP), :], mb)

        def pair(p, c2):
            for q in range(4):
                mb2[2 * p, pl.ds(q * 16, 16)] = mb[p, pl.ds(q * 16, 16)]
                mb2[2 * p + 1, pl.ds(q * 16, 16)] = mb[p, pl.ds(H + q * 16, 16)]
            return c2

        lax.fori_loop(0, CP, pair, 0)
        pltpu.sync_copy(mb2, agg_sh.at[idxd.at[j]], add=True)
        return carry

    lax.fori_loop(0, CH, chunk, 0)
    plsc.subcore_barrier()
    for t in range(SRS // CW):
        rows = pl.ds(sid * SRS + t * CW, CW)
        pltpu.sync_copy(agg_sh.at[rows, :], aggp.at[cid, rows, :])


@functools.cache
def _build_scatter_add():
    return pl.kernel(
        _scatter_add_body,
        out_type=jax.ShapeDtypeStruct((NC, NP, H2), _f32),
        mesh=_mesh(),
        scratch_types=[
            pltpu.VMEM((CH, CW), jnp.int32),
            pltpu.VMEM((CP, H2), _f32),
            pltpu.VMEM((CW, H2), _f32),
            pltpu.VMEM_SHARED((NP, H2), _f32),
        ],
    )


def _scatter_add(msg, dst3):
    return _build_scatter_add()(msg, dst3)


def _count_body(dst3, cntp, idxd, ones_b, zb, cnt_sh):
    cid = lax.axis_index("c")
    sid = lax.axis_index("s")
    wid = sid * NC + cid
    pltpu.sync_copy(dst3.at[wid], idxd)

    zz = jnp.zeros((16,), _f32)
    oo = jnp.ones((16,), _f32)

    def frow(r, c):
        for q in range(8):
            zb[r, pl.ds(q * 16, 16)] = zz
            ones_b[r, pl.ds(q * 16, 16)] = oo
        return c

    lax.fori_loop(0, CW, frow, 0)
    for t in range(SRS // CW):
        pltpu.sync_copy(zb, cnt_sh.at[pl.ds(sid * SRS + t * CW, CW), :])
    plsc.subcore_barrier()

    def chunk(j, carry):
        pltpu.sync_copy(ones_b, cnt_sh.at[idxd.at[j]], add=True)
        return carry

    lax.fori_loop(0, CH, chunk, 0)
    plsc.subcore_barrier()
    for t in range(SRS // CW):
        rows = pl.ds(sid * SRS + t * CW, CW)
        pltpu.sync_copy(cnt_sh.at[rows, :], cntp.at[cid, rows, :])


@functools.cache
def _build_count():
    return pl.kernel(
        _count_body,
        out_type=jax.ShapeDtypeStruct((NC, NP, H2), _f32),
        mesh=_mesh(),
        scratch_types=[
            pltpu.VMEM((CH, CW), jnp.int32),
            pltpu.VMEM((CW, H2), _f32),
            pltpu.VMEM((CW, H2), _f32),
            pltpu.VMEM_SHARED((NP, H2), _f32),
        ],
    )


def _count(dst3):
    return _build_count()(dst3)


def kernel(x, edge_index, edge_attr, params):
    src = edge_index[0].reshape(NW, CH, CW)
    dst = edge_index[1].reshape(NW, CH, CW)
    ea2 = edge_attr.reshape(E2, 2 * DE)

    def r1(v):
        return v.reshape(1, -1)

    pn = params['node_mlp']
    pe = params['edge_mlp']
    gnn = params['gnn']
    agg = params['agg']

    lp0 = gnn[0]
    node, t = _head(
        x, pn['W'], r1(pn['b']), r1(pn['g']), r1(pn['be']),
        lp0['mW1'][:, :H], r1(lp0['mb1']), lp0['mW1'][:, H:])

    u, st_e = _edge_u(ea2, _bdiag(pe['W']), _tile2(pe['b']))
    cntp = _count(dst)

    for li, lp in enumerate(gnn):
        h = _gather_add(t, dst, src)
        st_s = _hstats(h)
        msg = _msg(h, st_s, _bdiag(lp['mW2']), _tile2(lp['mb2']),
                   _tile2(lp['mg']), _tile2(lp['mbe']))
        aggp = _scatter_add(msg, dst)
        if li + 1 < len(gnn):
            nxt = gnn[li + 1]
            pwd, pbd, pws = nxt['mW1'][:, :H], r1(nxt['mb1']), nxt['mW1'][:, H:]
        else:
            aw1 = agg[0]['W']
            pwd = aw1[:, :H]
            pbd = jnp.zeros((1, H), _f32)
            pws = aw1[:, :H]
        node, t = _update(
            node, aggp, cntp, lp['uW1'][:, :H], lp['uW1'][:, H:],
            r1(lp['ub1']), lp['uW2'], r1(lp['ub2']), pwd, pbd, pws)

    d = _gather_diff(t, dst, src)
    t1, st1 = _t1(d, u, st_e, _tile2(pe['g']), _tile2(pe['be']),
                  _bdiag(agg[0]['W'][:, H:]), _tile2(agg[0]['b']))
    s2, st2 = _t2(t1, st1, _tile2(agg[0]['g']), _tile2(agg[0]['be']),
                  _bdiag(agg[1]['W']), _tile2(agg[1]['b']))
    y2 = _finout(s2, st2, _tile2(agg[1]['g']), _tile2(agg[1]['be']))
    return y2.reshape(E, H)


# speedup vs baseline: 3.5550x; 1.1261x over previous
import functools

import jax
import jax.numpy as jnp
from jax import lax
from jax.experimental import pallas as pl
from jax.experimental.pallas import tpu as pltpu
from jax.experimental.pallas import tpu_sc as plsc

N = 10000
E = 320000
E2 = E // 2
D = 128
DE = 16
H = 64
H2 = 2 * H
EPS = 1e-5

NC = 2
NS = 16
NW = NC * NS
EW = E // NW
CW = 80
CP = CW // 2
CH = EW // CW
NP = 10240
SRS = NP // NS
CE = 8000
GE = E2 // CE

_f32 = jnp.float32


def _swish(x):
    return x * (1.0 / (1.0 + jnp.exp(-x)))


def _mm_t(x, w):
    return lax.dot_general(x, w, (((1,), (1,)), ((), ())),
                           preferred_element_type=_f32)


def _bdiag(w):
    o, i = w.shape
    z = jnp.zeros((o, i), w.dtype)
    return jnp.concatenate(
        [jnp.concatenate([w, z], axis=1), jnp.concatenate([z, w], axis=1)],
        axis=0)


def _tile2(v):
    return jnp.concatenate([v, v]).reshape(1, -1)


def _bn_affine_paired(stats, g, be):
    ssum = stats[0:1, :H] + stats[0:1, H:]
    ssq = stats[1:2, :H] + stats[1:2, H:]
    m = ssum / E
    v = ssq / E - m * m
    scale = g[:, :H] / jnp.sqrt(v + EPS)
    shift = be[:, :H] - m * scale
    scale2 = jnp.concatenate([scale, scale], axis=1)
    shift2 = jnp.concatenate([shift, shift], axis=1)
    return scale2, shift2


def _head_body(x_ref, w_ref, b_ref, g_ref, be_ref, pwd_ref, pbd_ref, pws_ref,
               node_ref, t_ref):
    h = _swish(_mm_t(x_ref[...], w_ref[...]) + b_ref[...])
    m = jnp.mean(h, axis=0, keepdims=True)
    v = jnp.mean((h - m) * (h - m), axis=0, keepdims=True)
    node = g_ref[...] * (h - m) / jnp.sqrt(v + EPS) + be_ref[...]
    node_ref[...] = node
    ta = _mm_t(node, pwd_ref[...]) + pbd_ref[...]
    tb = _mm_t(node, pws_ref[...])
    t_ref[...] = jnp.concatenate([ta, tb], axis=1)


def _head(x, w, b, g, be, pwd, pbd, pws):
    out = [jax.ShapeDtypeStruct((N, H), _f32),
           jax.ShapeDtypeStruct((N, H2), _f32)]
    return pl.pallas_call(_head_body, out_shape=out)(
        x, w, b, g, be, pwd, pbd, pws)


def _edge_u_body(ea_ref, w_ref, b_ref, u_ref, st_ref, acc_ref):
    i = pl.program_id(0)
    s = _swish(_mm_t(ea_ref[...], w_ref[...]) + b_ref[...])
    u_ref[...] = s

    @pl.when(i == 0)
    def _():
        acc_ref[...] = jnp.zeros_like(acc_ref)

    acc_ref[0:1, :] += jnp.sum(s, axis=0, keepdims=True)
    acc_ref[1:2, :] += jnp.sum(s * s, axis=0, keepdims=True)

    @pl.when(i == pl.num_programs(0) - 1)
    def _():
        st_ref[...] = acc_ref[...]


def _edge_u(ea2, w2, b2):
    return pl.pallas_call(
        _edge_u_body,
        grid=(GE,),
        in_specs=[
            pl.BlockSpec((CE, 2 * DE), lambda i: (i, 0)),
            pl.BlockSpec((H2, 2 * DE), lambda i: (0, 0)),
            pl.BlockSpec((1, H2), lambda i: (0, 0)),
        ],
        out_specs=[
            pl.BlockSpec((CE, H2), lambda i: (i, 0)),
            pl.BlockSpec((2, H2), lambda i: (0, 0)),
        ],
        out_shape=[
            jax.ShapeDtypeStruct((E2, H2), _f32),
            jax.ShapeDtypeStruct((2, H2), _f32),
        ],
        scratch_shapes=[pltpu.VMEM((2, H2), _f32)],
    )(ea2, w2, b2)


def _hstats_body(h_ref, st_ref, acc_ref):
    i = pl.program_id(0)
    s = _swish(h_ref[...])

    @pl.when(i == 0)
    def _():
        acc_ref[...] = jnp.zeros_like(acc_ref)

    acc_ref[0:1, :] += jnp.sum(s, axis=0, keepdims=True)
    acc_ref[1:2, :] += jnp.sum(s * s, axis=0, keepdims=True)

    @pl.when(i == pl.num_programs(0) - 1)
    def _():
        st_ref[...] = acc_ref[...]


def _hstats(h):
    return pl.pallas_call(
        _hstats_body,
        grid=(GE,),
        in_specs=[pl.BlockSpec((CE, H2), lambda i: (i, 0))],
        out_specs=pl.BlockSpec((2, H2), lambda i: (0, 0)),
        out_shape=jax.ShapeDtypeStruct((2, H2), _f32),
        scratch_shapes=[pltpu.VMEM((2, H2), _f32)],
    )(h)


def _msg_body(h_ref, st_ref, w_ref, b_ref, g_ref, be_ref, out_ref):
    scale2, shift2 = _bn_affine_paired(st_ref[...], g_ref[...], be_ref[...])
    sn = _swish(h_ref[...]) * scale2 + shift2
    out_ref[...] = _swish(_mm_t(sn, w_ref[...]) + b_ref[...])


def _msg(h, st, w2, b2, g2, be2):
    return pl.pallas_call(
        _msg_body,
        grid=(GE,),
        in_specs=[
            pl.BlockSpec((CE, H2), lambda i: (i, 0)),
            pl.BlockSpec((2, H2), lambda i: (0, 0)),
            pl.BlockSpec((H2, H2), lambda i: (0, 0)),
            pl.BlockSpec((1, H2), lambda i: (0, 0)),
            pl.BlockSpec((1, H2), lambda i: (0, 0)),
            pl.BlockSpec((1, H2), lambda i: (0, 0)),
        ],
        out_specs=pl.BlockSpec((CE, H2), lambda i: (i, 0)),
        out_shape=jax.ShapeDtypeStruct((E2, H2), _f32),
    )(h, st, w2, b2, g2, be2)


def _update_body(node_ref, aggp_ref, uw1a_ref, uw1b_ref, ub1_ref,
                 uw2_ref, ub2_ref, pwd_ref, pbd_ref, pws_ref,
                 nn_ref, t_ref):
    node = node_ref[...]
    agg = aggp_ref[0, :N, :H] + aggp_ref[1, :N, :H]
    deg = aggp_ref[0, :N, H:H + 1] + aggp_ref[1, :N, H:H + 1]
    mean = agg / jnp.maximum(deg, 1.0)
    upd = _swish(_mm_t(node, uw1a_ref[...]) + _mm_t(mean, uw1b_ref[...])
                 + ub1_ref[...])
    upd = _swish(_mm_t(upd, uw2_ref[...]) + ub2_ref[...])
    nn = node + upd
    nn_ref[...] = nn
    ta = _mm_t(nn, pwd_ref[...]) + pbd_ref[...]
    tb = _mm_t(nn, pws_ref[...])
    t_ref[...] = jnp.concatenate([ta, tb], axis=1)


def _update(node, aggp, uw1a, uw1b, ub1, uw2, ub2, pwd, pbd, pws):
    out = [jax.ShapeDtypeStruct((N, H), _f32),
           jax.ShapeDtypeStruct((N, H2), _f32)]
    return pl.pallas_call(_update_body, out_shape=out)(
        node, aggp, uw1a, uw1b, ub1, uw2, ub2, pwd, pbd, pws)


def _t1_body(d_ref, u_ref, ste_ref, eg_ref, ebe_ref, w_ref, b_ref,
             t1_ref, st_ref, acc_ref):
    i = pl.program_id(0)
    scale2, shift2 = _bn_affine_paired(ste_ref[...], eg_ref[...], ebe_ref[...])
    edge = u_ref[...] * scale2 + shift2
    s = _swish(d_ref[...] + _mm_t(edge, w_ref[...]) + b_ref[...])
    t1_ref[...] = s

    @pl.when(i == 0)
    def _():
        acc_ref[...] = jnp.zeros_like(acc_ref)

    acc_ref[0:1, :] += jnp.sum(s, axis=0, keepdims=True)
    acc_ref[1:2, :] += jnp.sum(s * s, axis=0, keepdims=True)

    @pl.when(i == pl.num_programs(0) - 1)
    def _():
        st_ref[...] = acc_ref[...]


def _t1(d, u, ste, eg2, ebe2, w2, b2):
    return pl.pallas_call(
        _t1_body,
        grid=(GE,),
        in_specs=[
            pl.BlockSpec((CE, H2), lambda i: (i, 0)),
            pl.BlockSpec((CE, H2), lambda i: (i, 0)),
            pl.BlockSpec((2, H2), lambda i: (0, 0)),
            pl.BlockSpec((1, H2), lambda i: (0, 0)),
            pl.BlockSpec((1, H2), lambda i: (0, 0)),
            pl.BlockSpec((H2, H2), lambda i: (0, 0)),
            pl.BlockSpec((1, H2), lambda i: (0, 0)),
        ],
        out_specs=[
            pl.BlockSpec((CE, H2), lambda i: (i, 0)),
            pl.BlockSpec((2, H2), lambda i: (0, 0)),
        ],
        out_shape=[
            jax.ShapeDtypeStruct((E2, H2), _f32),
            jax.ShapeDtypeStruct((2, H2), _f32),
        ],
        scratch_shapes=[pltpu.VMEM((2, H2), _f32)],
    )(d, u, ste, eg2, ebe2, w2, b2)


def _t2_body(t1_ref, st1_ref, g1_ref, be1_ref, w_ref, b_ref,
             s2_ref, st_ref, acc_ref):
    i = pl.program_id(0)
    scale2, shift2 = _bn_affine_paired(st1_ref[...], g1_ref[...], be1_ref[...])
    tn = t1_ref[...] * scale2 + shift2
    s = _swish(_mm_t(tn, w_ref[...]) + b_ref[...])
    s2_ref[...] = s

    @pl.when(i == 0)
    def _():
        acc_ref[...] = jnp.zeros_like(acc_ref)

    acc_ref[0:1, :] += jnp.sum(s, axis=0, keepdims=True)
    acc_ref[1:2, :] += jnp.sum(s * s, axis=0, keepdims=True)

    @pl.when(i == pl.num_programs(0) - 1)
    def _():
        st_ref[...] = acc_ref[...]


def _t2(t1, st1, g12, be12, w2, b2):
    return pl.pallas_call(
        _t2_body,
        grid=(GE,),
        in_specs=[
            pl.BlockSpec((CE, H2), lambda i: (i, 0)),
            pl.BlockSpec((2, H2), lambda i: (0, 0)),
            pl.BlockSpec((1, H2), lambda i: (0, 0)),
            pl.BlockSpec((1, H2), lambda i: (0, 0)),
            pl.BlockSpec((H2, H2), lambda i: (0, 0)),
            pl.BlockSpec((1, H2), lambda i: (0, 0)),
        ],
        out_specs=[
            pl.BlockSpec((CE, H2), lambda i: (i, 0)),
            pl.BlockSpec((2, H2), lambda i: (0, 0)),
        ],
        out_shape=[
            jax.ShapeDtypeStruct((E2, H2), _f32),
            jax.ShapeDtypeStruct((2, H2), _f32),
        ],
        scratch_shapes=[pltpu.VMEM((2, H2), _f32)],
    )(t1, st1, g12, be12, w2, b2)


def _finout_body(s2_ref, st2_ref, g2_ref, be2_ref, y_ref):
    scale2, shift2 = _bn_affine_paired(st2_ref[...], g2_ref[...], be2_ref[...])
    y_ref[...] = s2_ref[...] * scale2 + shift2


def _finout(s2, st2, g22, be22):
    return pl.pallas_call(
        _finout_body,
        grid=(GE,),
        in_specs=[
            pl.BlockSpec((CE, H2), lambda i: (i, 0)),
            pl.BlockSpec((2, H2), lambda i: (0, 0)),
            pl.BlockSpec((1, H2), lambda i: (0, 0)),
            pl.BlockSpec((1, H2), lambda i: (0, 0)),
        ],
        out_specs=pl.BlockSpec((CE, H2), lambda i: (i, 0)),
        out_shape=jax.ShapeDtypeStruct((E2, H2), _f32),
    )(s2, st2, g22, be22)


@functools.cache
def _mesh():
    return plsc.VectorSubcoreMesh(core_axis_name="c", subcore_axis_name="s",
                                  num_cores=NC, num_subcores=NS)


def _worker_id():
    return lax.axis_index("s") * NC + lax.axis_index("c")


def _make_gather_body(diff):
    def body(t_hbm, dst_hbm, src_hbm, s_hbm,
             idxd, idxs, ra0, rb0, ra1, rb1, so0, so1, sg0, sg1, semo):
        wid = _worker_id()
        obase = wid * (EW // 2)
        pltpu.sync_copy(dst_hbm.at[wid], idxd)
        pltpu.sync_copy(src_hbm.at[wid], idxs)

        def fire(j, ra, rb, sg):
            pltpu.async_copy(t_hbm.at[idxd.at[j]], ra, sg)
            pltpu.async_copy(t_hbm.at[idxs.at[j]], rb, sg)

        def drain_g(ra, rb, sg):
            pltpu.make_async_copy(t_hbm.at[pl.ds(0, CW)], ra, sg).wait()
            pltpu.make_async_copy(t_hbm.at[pl.ds(0, CW)], rb, sg).wait()

        def drain_o(so):
            pltpu.make_async_copy(so, s_hbm.at[pl.ds(0, CP), :], semo).wait()

        def compute2(ra, rb, so):
            def pair(p, c):
                for half in range(2):
                    e = 2 * p + half
                    for q in range(4):
                        a = ra[e, pl.ds(q * 16, 16)]
                        b = rb[e, pl.ds(H + q * 16, 16)]
                        v = (a - b) if diff else (a + b)
                        so[p, pl.ds(half * H + q * 16, 16)] = v
                return c
            lax.fori_loop(0, CP, pair, 0)

        fire(0, ra0, rb0, sg0)

        def step(k, carry):
            j0 = 2 * k
            fire(j0 + 1, ra1, rb1, sg1)
            drain_g(ra0, rb0, sg0)

            @pl.when(j0 >= 2)
            def _():
                drain_o(so0)

            compute2(ra0, rb0, so0)
            pltpu.async_copy(so0, s_hbm.at[pl.ds(obase + j0 * CP, CP), :],
                             semo)
            jn = j0 + 2

            @pl.when(jn < CH)
            def _():
                fire(jn, ra0, rb0, sg0)

            drain_g(ra1, rb1, sg1)

            @pl.when(j0 + 1 >= 2)
            def _():
                drain_o(so1)

            compute2(ra1, rb1, so1)
            pltpu.async_copy(so1, s_hbm.at[pl.ds(obase + (j0 + 1) * CP, CP),
                                           :], semo)
            return carry

        lax.fori_loop(0, CH // 2, step, 0)
        jl = CH - 1
        drain_g(ra0, rb0, sg0)
        drain_o(so0)
        compute2(ra0, rb0, so0)
        pltpu.async_copy(so0, s_hbm.at[pl.ds(obase + jl * CP, CP), :], semo)
        drain_o(so1)
        drain_o(so0)

    return body


@functools.cache
def _build_gather_add():
    return pl.kernel(
        _make_gather_body(False),
        out_type=jax.ShapeDtypeStruct((E2, H2), _f32),
        mesh=_mesh(),
        scratch_types=[
            pltpu.VMEM((CH, CW), jnp.int32),
            pltpu.VMEM((CH, CW), jnp.int32),
            pltpu.VMEM((CW, H2), _f32),
            pltpu.VMEM((CW, H2), _f32),
            pltpu.VMEM((CW, H2), _f32),
            pltpu.VMEM((CW, H2), _f32),
            pltpu.VMEM((CP, H2), _f32),
            pltpu.VMEM((CP, H2), _f32),
            pltpu.SemaphoreType.DMA,
            pltpu.SemaphoreType.DMA,
            pltpu.SemaphoreType.DMA,
        ],
    )


def _gather_add(t, dst3, src3):
    return _build_gather_add()(t, dst3, src3)


@functools.cache
def _build_gather_diff():
    return pl.kernel(
        _make_gather_body(True),
        out_type=jax.ShapeDtypeStruct((E2, H2), _f32),
        mesh=_mesh(),
        scratch_types=[
            pltpu.VMEM((CH, CW), jnp.int32),
            pltpu.VMEM((CH, CW), jnp.int32),
            pltpu.VMEM((CW, H2), _f32),
            pltpu.VMEM((CW, H2), _f32),
            pltpu.VMEM((CW, H2), _f32),
            pltpu.VMEM((CW, H2), _f32),
            pltpu.VMEM((CP, H2), _f32),
            pltpu.VMEM((CP, H2), _f32),
            pltpu.SemaphoreType.DMA,
            pltpu.SemaphoreType.DMA,
            pltpu.SemaphoreType.DMA,
        ],
    )


def _gather_diff(t, dst3, src3):
    return _build_gather_diff()(t, dst3, src3)


def _scatter_add_body(msg, dst3, aggp, idxd, mb0, mb1, mb2a, mb2b, agg_sh,
                      semr):
    cid = lax.axis_index("c")
    sid = lax.axis_index("s")
    wid = sid * NC + cid
    ibase = wid * (EW // 2)
    pltpu.sync_copy(dst3.at[wid], idxd)

    zz = jnp.zeros((16,), _f32)
    one0 = jnp.where(lax.iota(jnp.int32, 16) == 0, 1.0, 0.0).astype(_f32)

    def zrow(r, c):
        for q in range(8):
            mb2a[r, pl.ds(q * 16, 16)] = zz
            mb2b[r, pl.ds(q * 16, 16)] = zz
        return c

    lax.fori_loop(0, CW, zrow, 0)
    for t in range(SRS // CW):
        pltpu.sync_copy(mb2a, agg_sh.at[pl.ds(sid * SRS + t * CW, CW), :])

    def crow(r, c):
        mb2a[r, pl.ds(H, 16)] = one0
        mb2b[r, pl.ds(H, 16)] = one0
        return c

    lax.fori_loop(0, CW, crow, 0)
    plsc.subcore_barrier()

    def fire(j, mb):
        pltpu.async_copy(msg.at[pl.ds(ibase + j * CP, CP), :], mb, semr)

    def drain(mb):
        pltpu.make_async_copy(msg.at[pl.ds(0, CP), :], mb, semr).wait()

    def unpack_scatter(j, mb, mb2):
        def pair(p, c2):
            for q in range(4):
                mb2[2 * p, pl.ds(q * 16, 16)] = mb[p, pl.ds(q * 16, 16)]
                mb2[2 * p + 1, pl.ds(q * 16, 16)] = mb[p, pl.ds(H + q * 16,
                                                                16)]
            return c2

        lax.fori_loop(0, CP, pair, 0)
        pltpu.sync_copy(mb2, agg_sh.at[idxd.at[j]], add=True)

    fire(0, mb0)

    def step(k, carry):
        j0 = 2 * k
        fire(j0 + 1, mb1)
        drain(mb0)
        unpack_scatter(j0, mb0, mb2a)
        jn = j0 + 2

        @pl.when(jn < CH)
        def _():
            fire(jn, mb0)

        drain(mb1)
        unpack_scatter(j0 + 1, mb1, mb2b)
        return carry

    lax.fori_loop(0, CH // 2, step, 0)
    drain(mb0)
    unpack_scatter(CH - 1, mb0, mb2a)

    plsc.subcore_barrier()
    for t in range(SRS // CW):
        rows = pl.ds(sid * SRS + t * CW, CW)
        pltpu.sync_copy(agg_sh.at[rows, :], aggp.at[cid, rows, :])


@functools.cache
def _build_scatter_add():
    return pl.kernel(
        _scatter_add_body,
        out_type=jax.ShapeDtypeStruct((NC, NP, H2), _f32),
        mesh=_mesh(),
        scratch_types=[
            pltpu.VMEM((CH, CW), jnp.int32),
            pltpu.VMEM((CP, H2), _f32),
            pltpu.VMEM((CP, H2), _f32),
            pltpu.VMEM((CW, H2), _f32),
            pltpu.VMEM((CW, H2), _f32),
            pltpu.VMEM_SHARED((NP, H2), _f32),
            pltpu.SemaphoreType.DMA,
        ],
    )


def _scatter_add(msg, dst3):
    return _build_scatter_add()(msg, dst3)


def kernel(x, edge_index, edge_attr, params):
    src = edge_index[0].reshape(NW, CH, CW)
    dst = edge_index[1].reshape(NW, CH, CW)
    ea2 = edge_attr.reshape(E2, 2 * DE)

    def r1(v):
        return v.reshape(1, -1)

    pn = params['node_mlp']
    pe = params['edge_mlp']
    gnn = params['gnn']
    agg = params['agg']

    lp0 = gnn[0]
    node, t = _head(
        x, pn['W'], r1(pn['b']), r1(pn['g']), r1(pn['be']),
        lp0['mW1'][:, :H], r1(lp0['mb1']), lp0['mW1'][:, H:])

    u, st_e = _edge_u(ea2, _bdiag(pe['W']), _tile2(pe['b']))

    for li, lp in enumerate(gnn):
        h = _gather_add(t, dst, src)
        st_s = _hstats(h)
        msg = _msg(h, st_s, _bdiag(lp['mW2']), _tile2(lp['mb2']),
                   _tile2(lp['mg']), _tile2(lp['mbe']))
        aggp = _scatter_add(msg, dst)
        if li + 1 < len(gnn):
            nxt = gnn[li + 1]
            pwd, pbd, pws = nxt['mW1'][:, :H], r1(nxt['mb1']), nxt['mW1'][:, H:]
        else:
            aw1 = agg[0]['W']
            pwd = aw1[:, :H]
            pbd = jnp.zeros((1, H), _f32)
            pws = aw1[:, :H]
        node, t = _update(
            node, aggp, lp['uW1'][:, :H], lp['uW1'][:, H:],
            r1(lp['ub1']), lp['uW2'], r1(lp['ub2']), pwd, pbd, pws)

    d = _gather_diff(t, dst, src)
    t1, st1 = _t1(d, u, st_e, _tile2(pe['g']), _tile2(pe['be']),
                  _bdiag(agg[0]['W'][:, H:]), _tile2(agg[0]['b']))
    s2, st2 = _t2(t1, st1, _tile2(agg[0]['g']), _tile2(agg[0]['be']),
                  _bdiag(agg[1]['W']), _tile2(agg[1]['b']))
    y2 = _finout(s2, st2, _tile2(agg[1]['g']), _tile2(agg[1]['be']))
    return y2.reshape(E, H)
